# baseline (speedup 1.0000x reference)
import functools

import jax
import jax.numpy as jnp
from jax.experimental import pallas as pl
from jax.experimental.pallas import tpu as pltpu

B, T, J, D = 2, 512, 86, 2
HID, FEAT, RDIM, NCLS = 128, 64, 256, 100
REGIONS = [(0, 21), (21, 42), (42, 61), (61, 86)]
DM = RDIM * len(REGIONS)
T2, T4 = T // 2, T // 4
BF = jnp.bfloat16


def _gelu(x):
    return 0.5 * x * (1.0 + jax.lax.erf(x * 0.7071067811865476))


def _pool_matrix(rows, cols):
    r = jax.lax.broadcasted_iota(jnp.int32, (rows, cols), 0)
    c = jax.lax.broadcasted_iota(jnp.int32, (rows, cols), 1)
    return jnp.where((c == 2 * r) | (c == 2 * r + 1), 0.5, 0.0).astype(jnp.float32)


def _mm(a, b):
    return jnp.dot(a, b, preferred_element_type=jnp.float32)


def _mm_nt(a, w):
    return jax.lax.dot_general(a, w, (((1,), (1,)), ((), ())),
                               preferred_element_type=jnp.float32)


def _region_stage(img_ref, regw, f_pad):
    x = img_ref[...]
    f_pad[0:2, :] = jnp.zeros((2, DM), BF)
    f_pad[T + 2:T + 6, :] = jnp.zeros((4, DM), BF)
    f_pad[T + T + 6:, :] = jnp.zeros((2, DM), BF)

    rowi = jax.lax.broadcasted_iota(jnp.int32, (J * D, HID), 0)
    for r, (s, e) in enumerate(REGIONS):
        W1, b1, W2T, b2, Wp, bp, g, be = regw[r]
        Jr = e - s
        jj, dd = rowi // 2, rowi % 2
        inreg = (jj >= s) & (jj < e)
        A = jnp.where(inreg, jnp.where(dd == 0, W1[0:1, :], W1[1:2, :]),
                      0.0) * (1.0 / Jr)
        h = jnp.maximum(_mm(x, A) + b1[0:1, :], 0.0)
        h = _mm_nt(h, W2T[...]) + b2[0:1, :]
        Wps = functools.reduce(
            lambda a, j: a + Wp[j * FEAT:(j + 1) * FEAT, :],
            range(1, Jr), Wp[0:FEAT, :])
        hp = _mm(h, Wps) + bp[0:1, :]
        m = jnp.mean(hp, axis=1, keepdims=True)
        v = jnp.mean((hp - m) ** 2, axis=1, keepdims=True)
        fr = _gelu((hp - m) * jax.lax.rsqrt(v + 1e-5) * g[0:1, :]
                   + be[0:1, :]).astype(BF)
        f_pad[2:T + 2, r * RDIM:(r + 1) * RDIM] = fr[0:T, :]
        f_pad[T + 6:T + T + 6, r * RDIM:(r + 1) * RDIM] = fr[T:, :]


def _body(img_ref,
          rh_W1, rh_b1, rh_W2, rh_b2, rh_Wp, rh_bp, rh_g, rh_be,
          lh_W1, lh_b1, lh_W2, lh_b2, lh_Wp, lh_bp, lh_g, lh_be,
          lp_W1, lp_b1, lp_W2, lp_b2, lp_Wp, lp_bp, lp_g, lp_be,
          bd_W1, bd_b1, bd_W2, bd_b2, bd_Wp, bd_bp, bd_g, bd_be,
          cw1, cw2, cw3, tc1_b, tc2_b, sc_b, out_WT, out_b,
          out_ref, f_pad, p1_pad, in_pad, acc1, acc2, acc3):
    i = pl.program_id(0)
    regw = [(rh_W1, rh_b1, rh_W2, rh_b2, rh_Wp, rh_bp, rh_g, rh_be),
            (lh_W1, lh_b1, lh_W2, lh_b2, lh_Wp, lh_bp, lh_g, lh_be),
            (lp_W1, lp_b1, lp_W2, lp_b2, lp_Wp, lp_bp, lp_g, lp_be),
            (bd_W1, bd_b1, bd_W2, bd_b2, bd_Wp, bd_bp, bd_g, bd_be)]

    @pl.when(i == 0)
    def _():
        _region_stage(img_ref, regw, f_pad)

    for s in range(5):
        @pl.when(i == s)
        def _(s=s):
            w = cw1[0].astype(BF)
            for b in range(B):
                sl = f_pad[b * (T + 4) + s:b * (T + 4) + s + T, :]
                if s == 0:
                    acc1[b * T:(b + 1) * T, :] = _mm_nt(sl, w)
                else:
                    acc1[b * T:(b + 1) * T, :] += _mm_nt(sl, w)

    @pl.when(i == 4)
    def _():
        P1 = _pool_matrix(T2, T)
        p1_pad[0:2, :] = jnp.zeros((2, DM), BF)
        p1_pad[T2 + 2:T2 + 6, :] = jnp.zeros((4, DM), BF)
        p1_pad[T2 + T2 + 6:, :] = jnp.zeros((2, DM), BF)
        for b in range(B):
            g1 = _gelu(acc1[b * T:(b + 1) * T, :] + tc1_b[0:1, :])
            o2 = b * (T2 + 4)
            p1_pad[o2 + 2:o2 + 2 + T2, :] = _mm(P1, g1).astype(BF)

    for s in range(5, 10):
        @pl.when(i == s)
        def _(s=s):
            w = cw2[0].astype(BF)
            k = s - 5
            for b in range(B):
                sl = p1_pad[b * (T2 + 4) + k:b * (T2 + 4) + k + T2, :]
                if k == 0:
                    acc2[b * T2:(b + 1) * T2, :] = _mm_nt(sl, w)
                else:
                    acc2[b * T2:(b + 1) * T2, :] += _mm_nt(sl, w)

    @pl.when(i == 9)
    def _():
        P2 = _pool_matrix(T4, T2)
        in_pad[0:1, :] = jnp.zeros((1, DM), BF)
        in_pad[T4 + 1:T4 + 3, :] = jnp.zeros((2, DM), BF)
        in_pad[T4 + T4 + 3:, :] = jnp.zeros((1, DM), BF)
        for b in range(B):
            g2 = _gelu(acc2[b * T2:(b + 1) * T2, :] + tc2_b[0:1, :])
            o3 = b * (T4 + 2)
            in_pad[o3 + 1:o3 + 1 + T4, :] = _mm(P2, g2).astype(BF)

    for s in range(10, 13):
        @pl.when(i == s)
        def _(s=s):
            w = cw3[0].astype(BF)
            k = s - 10
            for b in range(B):
                sl = in_pad[b * (T4 + 2) + k:b * (T4 + 2) + k + T4, :]
                if k == 0:
                    acc3[b * T4:(b + 1) * T4, :] = _mm_nt(sl, w)
                else:
                    acc3[b * T4:(b + 1) * T4, :] += _mm_nt(sl, w)

    @pl.when(i == 12)
    def _():
        for b in range(B):
            sec = _gelu(acc3[b * T4:(b + 1) * T4, :] + sc_b[0:1, :])
            out_ref[b, :, :] = _mm_nt(sec, out_WT[...]) + out_b[0:1, :]


def _const(a):
    return pl.BlockSpec(a.shape, lambda i: (0,) * a.ndim)


@jax.jit
def _run(img, *ws):
    *reg_ws, cw1, cw2, cw3, tc1_b, tc2_b, sc_b, out_WT, out_b = ws
    specs = ([_const(img)] + [_const(w) for w in reg_ws]
             + [pl.BlockSpec((1, DM, DM),
                             lambda i: (jnp.minimum(i, 4), 0, 0)),
                pl.BlockSpec((1, DM, DM),
                             lambda i: (jnp.clip(i - 5, 0, 4), 0, 0)),
                pl.BlockSpec((1, DM, DM),
                             lambda i: (jnp.clip(i - 10, 0, 2), 0, 0)),
                _const(tc1_b), _const(tc2_b), _const(sc_b),
                _const(out_WT), _const(out_b)])
    return pl.pallas_call(
        _body,
        grid=(13,),
        out_shape=jax.ShapeDtypeStruct((B, T4, NCLS), jnp.float32),
        in_specs=specs,
        out_specs=pl.BlockSpec((B, T4, NCLS), lambda i: (0, 0, 0)),
        scratch_shapes=[
            pltpu.VMEM((B * (T + 4), DM), BF),
            pltpu.VMEM((B * (T2 + 4), DM), BF),
            pltpu.VMEM((B * (T4 + 2), DM), BF),
            pltpu.VMEM((B * T, DM), jnp.float32),
            pltpu.VMEM((B * T2, DM), jnp.float32),
            pltpu.VMEM((B * T4, DM), jnp.float32),
        ],
        compiler_params=pltpu.CompilerParams(
            vmem_limit_bytes=60 * 1024 * 1024),
    )(img, *ws)


def kernel(tokenizer, images,
           rh_W1, rh_b1, rh_W2, rh_b2, rh_Wp, rh_bp, rh_g, rh_be,
           lh_W1, lh_b1, lh_W2, lh_b2, lh_Wp, lh_bp, lh_g, lh_be,
           lp_W1, lp_b1, lp_W2, lp_b2, lp_Wp, lp_bp, lp_g, lp_be,
           bd_W1, bd_b1, bd_W2, bd_b2, bd_Wp, bd_bp, bd_g, bd_be,
           tc1_w, tc1_b, tc2_w, tc2_b, sc_w, sc_b,
           mlp_W, mlp_b, out_W, out_b):
    img = images.reshape(B * T, J * D)
    r2 = lambda v: v.reshape(1, -1)
    tr = jnp.transpose
    cw1, cw2, cw3 = tr(tc1_w, (2, 0, 1)), tr(tc2_w, (2, 0, 1)), tr(sc_w, (2, 0, 1))
    ws = (rh_W1, r2(rh_b1), tr(rh_W2), r2(rh_b2), rh_Wp, r2(rh_bp), r2(rh_g), r2(rh_be),
          lh_W1, r2(lh_b1), tr(lh_W2), r2(lh_b2), lh_Wp, r2(lh_bp), r2(lh_g), r2(lh_be),
          lp_W1, r2(lp_b1), tr(lp_W2), r2(lp_b2), lp_Wp, r2(lp_bp), r2(lp_g), r2(lp_be),
          bd_W1, r2(bd_b1), tr(bd_W2), r2(bd_b2), bd_Wp, r2(bd_bp), r2(bd_g), r2(bd_be),
          cw1, cw2, cw3, r2(tc1_b), r2(tc2_b), r2(sc_b),
          tr(out_W), r2(out_b))
    return _run(img, *ws)


# speedup vs baseline: 1.0130x; 1.0130x over previous
import functools

import jax
import jax.numpy as jnp
from jax.experimental import pallas as pl
from jax.experimental.pallas import tpu as pltpu

B, T, J, D = 2, 512, 86, 2
HID, FEAT, RDIM, NCLS = 128, 64, 256, 100
REGIONS = [(0, 21), (21, 42), (42, 61), (61, 86)]
DM = RDIM * len(REGIONS)
T2, T4 = T // 2, T // 4
BF = jnp.bfloat16


def _gelu(x):
    return 0.5 * x * (1.0 + jax.lax.erf(x * 0.7071067811865476))


def _pool_matrix(rows, cols):
    r = jax.lax.broadcasted_iota(jnp.int32, (rows, cols), 0)
    c = jax.lax.broadcasted_iota(jnp.int32, (rows, cols), 1)
    return jnp.where((c == 2 * r) | (c == 2 * r + 1), 0.5, 0.0).astype(jnp.float32)


def _mm(a, b):
    return jnp.dot(a, b, preferred_element_type=jnp.float32)


def _mm_nt(a, w):
    return jax.lax.dot_general(a, w, (((1,), (1,)), ((), ())),
                               preferred_element_type=jnp.float32)


def _region_stage(img_ref, regw, f_pad):
    x = img_ref[...]
    f_pad[0:2, :] = jnp.zeros((2, DM), BF)
    f_pad[T + 2:T + 6, :] = jnp.zeros((4, DM), BF)
    f_pad[T + T + 6:, :] = jnp.zeros((2, DM), BF)

    rowi = jax.lax.broadcasted_iota(jnp.int32, (J * D, HID), 0)
    for r, (s, e) in enumerate(REGIONS):
        W1, b1, W2T, b2, Wp, bp, g, be = regw[r]
        Jr = e - s
        jj, dd = rowi // 2, rowi % 2
        inreg = (jj >= s) & (jj < e)
        A = jnp.where(inreg, jnp.where(dd == 0, W1[0:1, :], W1[1:2, :]),
                      0.0) * (1.0 / Jr)
        h = jnp.maximum(_mm(x, A) + b1[0:1, :], 0.0)
        h = _mm_nt(h, W2T[...]) + b2[0:1, :]
        Wps = functools.reduce(
            lambda a, j: a + Wp[j * FEAT:(j + 1) * FEAT, :],
            range(1, Jr), Wp[0:FEAT, :])
        hp = _mm(h, Wps) + bp[0:1, :]
        m = jnp.mean(hp, axis=1, keepdims=True)
        v = jnp.mean((hp - m) ** 2, axis=1, keepdims=True)
        fr = _gelu((hp - m) * jax.lax.rsqrt(v + 1e-5) * g[0:1, :]
                   + be[0:1, :]).astype(BF)
        f_pad[2:T + 2, r * RDIM:(r + 1) * RDIM] = fr[0:T, :]
        f_pad[T + 6:T + T + 6, r * RDIM:(r + 1) * RDIM] = fr[T:, :]


def _body(img_ref,
          rh_W1, rh_b1, rh_W2, rh_b2, rh_Wp, rh_bp, rh_g, rh_be,
          lh_W1, lh_b1, lh_W2, lh_b2, lh_Wp, lh_bp, lh_g, lh_be,
          lp_W1, lp_b1, lp_W2, lp_b2, lp_Wp, lp_bp, lp_g, lp_be,
          bd_W1, bd_b1, bd_W2, bd_b2, bd_Wp, bd_bp, bd_g, bd_be,
          cw1, cw2, cw3, tc1_b, tc2_b, sc_b, out_WT, out_b,
          out_ref, f_pad, p1_pad, in_pad, acc1, acc2, acc3):
    i = pl.program_id(0)
    regw = [(rh_W1, rh_b1, rh_W2, rh_b2, rh_Wp, rh_bp, rh_g, rh_be),
            (lh_W1, lh_b1, lh_W2, lh_b2, lh_Wp, lh_bp, lh_g, lh_be),
            (lp_W1, lp_b1, lp_W2, lp_b2, lp_Wp, lp_bp, lp_g, lp_be),
            (bd_W1, bd_b1, bd_W2, bd_b2, bd_Wp, bd_bp, bd_g, bd_be)]

    @pl.when(i == 0)
    def _():
        _region_stage(img_ref, regw, f_pad)

    for s in range(5):
        @pl.when(i == s)
        def _(s=s):
            w = cw1[0].astype(BF)
            sl = f_pad[s:s + 2 * T + 4, :]
            if s == 0:
                acc1[...] = _mm_nt(sl, w)
            else:
                acc1[...] += _mm_nt(sl, w)

    @pl.when(i == 4)
    def _():
        P1 = _pool_matrix(T2, T)
        p1_pad[0:2, :] = jnp.zeros((2, DM), BF)
        p1_pad[T2 + 2:T2 + 6, :] = jnp.zeros((4, DM), BF)
        p1_pad[T2 + T2 + 6:, :] = jnp.zeros((2, DM), BF)
        for b in range(B):
            g1 = _gelu(acc1[b * (T + 4):b * (T + 4) + T, :] + tc1_b[0:1, :])
            o2 = b * (T2 + 4)
            p1_pad[o2 + 2:o2 + 2 + T2, :] = _mm(P1, g1).astype(BF)

    for s in range(5, 10):
        @pl.when(i == s)
        def _(s=s):
            w = cw2[0].astype(BF)
            k = s - 5
            sl = p1_pad[k:k + 2 * T2 + 4, :]
            if k == 0:
                acc2[...] = _mm_nt(sl, w)
            else:
                acc2[...] += _mm_nt(sl, w)

    @pl.when(i == 9)
    def _():
        P2 = _pool_matrix(T4, T2)
        in_pad[0:1, :] = jnp.zeros((1, DM), BF)
        in_pad[T4 + 1:T4 + 3, :] = jnp.zeros((2, DM), BF)
        in_pad[T4 + T4 + 3:, :] = jnp.zeros((1, DM), BF)
        for b in range(B):
            g2 = _gelu(acc2[b * (T2 + 4):b * (T2 + 4) + T2, :] + tc2_b[0:1, :])
            o3 = b * (T4 + 2)
            in_pad[o3 + 1:o3 + 1 + T4, :] = _mm(P2, g2).astype(BF)

    for s in range(10, 13):
        @pl.when(i == s)
        def _(s=s):
            w = cw3[0].astype(BF)
            k = s - 10
            sl = in_pad[k:k + 2 * T4 + 2, :]
            if k == 0:
                acc3[...] = _mm_nt(sl, w)
            else:
                acc3[...] += _mm_nt(sl, w)

    @pl.when(i == 12)
    def _():
        for b in range(B):
            sec = _gelu(acc3[b * (T4 + 2):b * (T4 + 2) + T4, :] + sc_b[0:1, :])
            out_ref[b, :, :] = _mm_nt(sec, out_WT[...]) + out_b[0:1, :]


def _const(a):
    return pl.BlockSpec(a.shape, lambda i: (0,) * a.ndim)


@jax.jit
def _run(img, *ws):
    *reg_ws, cw1, cw2, cw3, tc1_b, tc2_b, sc_b, out_WT, out_b = ws
    specs = ([_const(img)] + [_const(w) for w in reg_ws]
             + [pl.BlockSpec((1, DM, DM),
                             lambda i: (jnp.minimum(i, 4), 0, 0)),
                pl.BlockSpec((1, DM, DM),
                             lambda i: (jnp.clip(i - 5, 0, 4), 0, 0)),
                pl.BlockSpec((1, DM, DM),
                             lambda i: (jnp.clip(i - 10, 0, 2), 0, 0)),
                _const(tc1_b), _const(tc2_b), _const(sc_b),
                _const(out_WT), _const(out_b)])
    return pl.pallas_call(
        _body,
        grid=(13,),
        out_shape=jax.ShapeDtypeStruct((B, T4, NCLS), jnp.float32),
        in_specs=specs,
        out_specs=pl.BlockSpec((B, T4, NCLS), lambda i: (0, 0, 0)),
        scratch_shapes=[
            pltpu.VMEM((B * (T + 4), DM), BF),
            pltpu.VMEM((B * (T2 + 4), DM), BF),
            pltpu.VMEM((B * (T4 + 2), DM), BF),
            pltpu.VMEM((2 * T + 4, DM), jnp.float32),
            pltpu.VMEM((2 * T2 + 4, DM), jnp.float32),
            pltpu.VMEM((2 * T4 + 2, DM), jnp.float32),
        ],
        compiler_params=pltpu.CompilerParams(
            vmem_limit_bytes=60 * 1024 * 1024),
    )(img, *ws)


def kernel(tokenizer, images,
           rh_W1, rh_b1, rh_W2, rh_b2, rh_Wp, rh_bp, rh_g, rh_be,
           lh_W1, lh_b1, lh_W2, lh_b2, lh_Wp, lh_bp, lh_g, lh_be,
           lp_W1, lp_b1, lp_W2, lp_b2, lp_Wp, lp_bp, lp_g, lp_be,
           bd_W1, bd_b1, bd_W2, bd_b2, bd_Wp, bd_bp, bd_g, bd_be,
           tc1_w, tc1_b, tc2_w, tc2_b, sc_w, sc_b,
           mlp_W, mlp_b, out_W, out_b):
    img = images.reshape(B * T, J * D)
    r2 = lambda v: v.reshape(1, -1)
    tr = jnp.transpose
    cw1, cw2, cw3 = tr(tc1_w, (2, 0, 1)), tr(tc2_w, (2, 0, 1)), tr(sc_w, (2, 0, 1))
    ws = (rh_W1, r2(rh_b1), tr(rh_W2), r2(rh_b2), rh_Wp, r2(rh_bp), r2(rh_g), r2(rh_be),
          lh_W1, r2(lh_b1), tr(lh_W2), r2(lh_b2), lh_Wp, r2(lh_bp), r2(lh_g), r2(lh_be),
          lp_W1, r2(lp_b1), tr(lp_W2), r2(lp_b2), lp_Wp, r2(lp_bp), r2(lp_g), r2(lp_be),
          bd_W1, r2(bd_b1), tr(bd_W2), r2(bd_b2), bd_Wp, r2(bd_bp), r2(bd_g), r2(bd_be),
          cw1, cw2, cw3, r2(tc1_b), r2(tc2_b), r2(sc_b),
          tr(out_W), r2(out_b))
    return _run(img, *ws)
